# baseline (speedup 1.0000x reference)
import functools

import jax
import jax.numpy as jnp
from jax.experimental import pallas as pl
from jax.experimental.pallas import tpu as pltpu

L = 2048
D = 1024
H = 16
DH = 64
DREL = 64
KK = 512
BQ = 256
BK = 256
NB = L // BQ

_INT_MIN = -2147483648
_NEG = -1e30

_DN_TT = (((1,), (1,)), ((), ()))
_DN_NN = (((1,), (0,)), ((), ()))


def _fused_body(hs, rel, wq, wk, wv, wqr, wkr, wo, out,
                qh_s, kh_s, vh_s, rk_s, bias_ref, sm_ref, acc):
    qb = pl.program_id(0)

    @pl.when(qb == 0)
    def _prologue():
        h = hs[...]
        qh_s[...] = (jax.lax.dot_general(
            h, wq[...], _DN_TT,
            preferred_element_type=jnp.float32)
            * (DH ** -0.5)).astype(jnp.bfloat16)
        kh_s[...] = jax.lax.dot_general(
            h, wk[...], _DN_TT,
            preferred_element_type=jnp.float32).astype(jnp.bfloat16)
        vh_s[...] = jax.lax.dot_general(
            h, wv[...], _DN_TT,
            preferred_element_type=jnp.float32).astype(jnp.bfloat16)
        rk_s[...] = jax.lax.dot_general(
            rel[...], wkr[...], _DN_TT,
            preferred_element_type=jnp.float32)

    qs = pl.ds(qb * BQ, BQ)
    rq = jax.lax.dot_general(rel[qs, :], wqr[...], _DN_TT,
                             preferred_element_type=jnp.float32) \
        * (DREL ** -0.5)
    scores = jax.lax.dot_general(rq, rk_s[...], _DN_TT,
                                 preferred_element_type=jnp.float32)
    rows = qb * BQ + jax.lax.broadcasted_iota(jnp.int32, (BQ, L), 0)
    cols = jax.lax.broadcasted_iota(jnp.int32, (BQ, L), 1)
    causal = cols <= rows
    bits = jax.lax.bitcast_convert_type(scores, jnp.int32)
    mp = jnp.where(bits >= 0, bits, bits ^ jnp.int32(0x7FFFFFFF))
    mp = jnp.where(causal, mp, jnp.int32(_INT_MIN))
    cnt = jnp.sum((mp >= 0).astype(jnp.int32), axis=1, keepdims=True)
    t = jnp.where(cnt >= KK, jnp.int32(0), jnp.int32(_INT_MIN))
    for b in range(30, 6, -1):
        cand = t | jnp.int32(1 << b)
        cnt = jnp.sum((mp >= cand).astype(jnp.int32), axis=1, keepdims=True)
        t = jnp.where(cnt >= KK, cand, t)
    t = jnp.maximum(t, jnp.int32(_INT_MIN + 1))
    allowed = (mp >= t) | (cols == rows)
    bias_ref[...] = jnp.where(allowed, jnp.float32(0.0), jnp.float32(_NEG))

    sm_ref[...] = jnp.zeros((BQ, 128), jnp.float32)
    acc[...] = jnp.zeros((BQ, D), jnp.float32)

    for c in range(NB):
        @pl.when(c <= qb)
        def _chunk(c=c):
            ks = pl.ds(c * BK, BK)
            b_c = bias_ref[:, ks]
            for h in range(H):
                sl = slice(h * DH, (h + 1) * DH)
                s = jax.lax.dot_general(
                    qh_s[qs, sl], kh_s[ks, sl], _DN_TT,
                    preferred_element_type=jnp.float32) + b_c
                p = jnp.exp(s)
                sm_ref[:, h:h + 1] += jnp.sum(p, axis=1, keepdims=True)
                acc[:, sl] += jax.lax.dot_general(
                    p.astype(jnp.bfloat16), vh_s[ks, sl], _DN_NN,
                    preferred_element_type=jnp.float32)

    for h in range(H):
        sl = slice(h * DH, (h + 1) * DH)
        acc[:, sl] = acc[:, sl] / sm_ref[:, h:h + 1]
    out[...] = jax.lax.dot_general(
        acc[...].astype(jnp.bfloat16), wo[...], _DN_TT,
        preferred_element_type=jnp.float32)


@jax.jit
def _run(hs, rel, wqr, wkr, wq, wk, wv, wo):
    out = pl.pallas_call(
        _fused_body,
        grid=(NB,),
        compiler_params=pltpu.CompilerParams(
            dimension_semantics=("arbitrary",)),
        in_specs=[
            pl.BlockSpec((L, D), lambda i: (0, 0)),
            pl.BlockSpec((L, D), lambda i: (0, 0)),
            pl.BlockSpec((D, D), lambda i: (0, 0)),
            pl.BlockSpec((D, D), lambda i: (0, 0)),
            pl.BlockSpec((D, D), lambda i: (0, 0)),
            pl.BlockSpec((DREL, D), lambda i: (0, 0)),
            pl.BlockSpec((DREL, D), lambda i: (0, 0)),
            pl.BlockSpec((D, D), lambda i: (0, 0)),
        ],
        out_specs=pl.BlockSpec((BQ, D), lambda i: (i, 0)),
        out_shape=jax.ShapeDtypeStruct((L, D), jnp.float32),
        scratch_shapes=[
            pltpu.VMEM((L, D), jnp.bfloat16),
            pltpu.VMEM((L, D), jnp.bfloat16),
            pltpu.VMEM((L, D), jnp.bfloat16),
            pltpu.VMEM((L, DREL), jnp.float32),
            pltpu.VMEM((BQ, L), jnp.float32),
            pltpu.VMEM((BQ, 128), jnp.float32),
            pltpu.VMEM((BQ, D), jnp.float32),
        ],
    )(hs, rel, wq, wk, wv, wqr, wkr, wo.astype(jnp.bfloat16))
    return out


def kernel(hidden_states, relevance, W_q_rel, W_k_rel, Wq, Wk, Wv, Wo):
    hs = hidden_states.reshape(L, D)
    rel = relevance.reshape(L, D)
    out = _run(hs, rel, W_q_rel, W_k_rel, Wq, Wk, Wv, Wo)
    return out.reshape(1, L, D)


# speedup vs baseline: 1.8601x; 1.8601x over previous
import functools

import jax
import jax.numpy as jnp
from jax.experimental import pallas as pl
from jax.experimental.pallas import tpu as pltpu

L = 2048
D = 1024
H = 16
DH = 64
DREL = 64
KK = 512
BQ = 256
BK = 256
NB = L // BQ

_INT_MIN = -2147483648
_NEG = -1e30

_DN_TT = (((1,), (1,)), ((), ()))
_DN_NN = (((1,), (0,)), ((), ()))


def _projmask_body(hid, rel, rel_full, wq, wk, wv, wqr, wkr,
                   qh, kh, vh, biasb, rk_s):
    i = pl.program_id(0)

    @pl.when(i == 0)
    def _rk():
        rk_s[...] = jax.lax.dot_general(
            rel_full[...], wkr[...], _DN_TT,
            preferred_element_type=jnp.float32)

    h = hid[...]
    qh[...] = (jax.lax.dot_general(
        h, wq[...], _DN_TT,
        preferred_element_type=jnp.float32)
        * (DH ** -0.5)).astype(jnp.bfloat16)
    kh[...] = jax.lax.dot_general(
        h, wk[...], _DN_TT,
        preferred_element_type=jnp.float32).astype(jnp.bfloat16)
    vh[...] = jax.lax.dot_general(
        h, wv[...], _DN_TT,
        preferred_element_type=jnp.float32).astype(jnp.bfloat16)

    rq = jax.lax.dot_general(rel[...], wqr[...], _DN_TT,
                             preferred_element_type=jnp.float32) \
        * (DREL ** -0.5)
    scores = jax.lax.dot_general(rq, rk_s[...], _DN_TT,
                                 preferred_element_type=jnp.float32)
    rows = i * BQ + jax.lax.broadcasted_iota(jnp.int32, (BQ, L), 0)
    cols = jax.lax.broadcasted_iota(jnp.int32, (BQ, L), 1)
    causal = cols <= rows
    bits = jax.lax.bitcast_convert_type(scores, jnp.int32)
    mp = jnp.where(bits >= 0, bits, bits ^ jnp.int32(0x7FFFFFFF))
    mp = jnp.where(causal, mp, jnp.int32(_INT_MIN))
    cnt = jnp.sum((mp >= 0).astype(jnp.int32), axis=1, keepdims=True)
    t = jnp.where(cnt >= KK, jnp.int32(0), jnp.int32(_INT_MIN))
    for b in range(30, 6, -1):
        cand = t | jnp.int32(1 << b)
        cnt = jnp.sum((mp >= cand).astype(jnp.int32), axis=1, keepdims=True)
        t = jnp.where(cnt >= KK, cand, t)
    t = jnp.maximum(t, jnp.int32(_INT_MIN + 1))
    allowed = (mp >= t) | (cols == rows)
    biasb[...] = jnp.where(allowed, jnp.float32(0.0),
                           jnp.float32(_NEG)).astype(jnp.bfloat16)


def _attn_body(qh, kh, vh, biasb, wo, out, sm_ref, acc):
    qb = pl.program_id(0)

    sm_ref[...] = jnp.zeros((BQ, 128), jnp.float32)
    acc[...] = jnp.zeros((BQ, D), jnp.float32)

    for c in range(NB):
        @pl.when(c <= qb)
        def _chunk(c=c):
            ks = pl.ds(c * BK, BK)
            b_c = biasb[:, ks].astype(jnp.float32)
            for h in range(H):
                sl = slice(h * DH, (h + 1) * DH)
                s = jax.lax.dot_general(
                    qh[:, sl], kh[ks, sl], _DN_TT,
                    preferred_element_type=jnp.float32) + b_c
                p = jnp.exp(s)
                sm_ref[:, h:h + 1] += jnp.sum(p, axis=1, keepdims=True)
                acc[:, sl] += jax.lax.dot_general(
                    p.astype(jnp.bfloat16), vh[ks, sl], _DN_NN,
                    preferred_element_type=jnp.float32)

    for h in range(H):
        sl = slice(h * DH, (h + 1) * DH)
        acc[:, sl] = acc[:, sl] / sm_ref[:, h:h + 1]
    out[...] = jax.lax.dot_general(
        acc[...].astype(jnp.bfloat16), wo[...], _DN_TT,
        preferred_element_type=jnp.float32)


@jax.jit
def _run(hs, rel, wqr, wkr, wq, wk, wv, wo):
    qh, kh, vh, biasb = pl.pallas_call(
        _projmask_body,
        grid=(NB,),
        compiler_params=pltpu.CompilerParams(
            dimension_semantics=("arbitrary",)),
        in_specs=[
            pl.BlockSpec((BQ, D), lambda i: (i, 0)),
            pl.BlockSpec((BQ, D), lambda i: (i, 0)),
            pl.BlockSpec((L, D), lambda i: (0, 0)),
            pl.BlockSpec((D, D), lambda i: (0, 0)),
            pl.BlockSpec((D, D), lambda i: (0, 0)),
            pl.BlockSpec((D, D), lambda i: (0, 0)),
            pl.BlockSpec((DREL, D), lambda i: (0, 0)),
            pl.BlockSpec((DREL, D), lambda i: (0, 0)),
        ],
        out_specs=[
            pl.BlockSpec((BQ, D), lambda i: (i, 0)),
            pl.BlockSpec((BQ, D), lambda i: (i, 0)),
            pl.BlockSpec((BQ, D), lambda i: (i, 0)),
            pl.BlockSpec((BQ, L), lambda i: (i, 0)),
        ],
        out_shape=[
            jax.ShapeDtypeStruct((L, D), jnp.bfloat16),
            jax.ShapeDtypeStruct((L, D), jnp.bfloat16),
            jax.ShapeDtypeStruct((L, D), jnp.bfloat16),
            jax.ShapeDtypeStruct((L, L), jnp.bfloat16),
        ],
        scratch_shapes=[
            pltpu.VMEM((L, DREL), jnp.float32),
        ],
    )(hs, rel, rel, wq, wk, wv, wqr, wkr)

    out = pl.pallas_call(
        _attn_body,
        grid=(NB,),
        compiler_params=pltpu.CompilerParams(
            dimension_semantics=("parallel",)),
        in_specs=[
            pl.BlockSpec((BQ, D), lambda i: (i, 0)),
            pl.BlockSpec((L, D), lambda i: (0, 0)),
            pl.BlockSpec((L, D), lambda i: (0, 0)),
            pl.BlockSpec((BQ, L), lambda i: (i, 0)),
            pl.BlockSpec((D, D), lambda i: (0, 0)),
        ],
        out_specs=pl.BlockSpec((BQ, D), lambda i: (i, 0)),
        out_shape=jax.ShapeDtypeStruct((L, D), jnp.float32),
        scratch_shapes=[
            pltpu.VMEM((BQ, 128), jnp.float32),
            pltpu.VMEM((BQ, D), jnp.float32),
        ],
    )(qh, kh, vh, biasb, wo.astype(jnp.bfloat16))
    return out


def kernel(hidden_states, relevance, W_q_rel, W_k_rel, Wq, Wk, Wv, Wo):
    hs = hidden_states.reshape(L, D)
    rel = relevance.reshape(L, D)
    out = _run(hs, rel, W_q_rel, W_k_rel, Wq, Wk, Wv, Wo)
    return out.reshape(1, L, D)
